# baseline (speedup 1.0000x reference)
import functools

import jax
import jax.numpy as jnp
from jax import lax
from jax.experimental import pallas as pl
from jax.experimental.pallas import tpu as pltpu
from jax.experimental.pallas import tpu_sc as plsc

B = 16384
D = 64
NUM_REL = 1000
TP = 1024

_info = plsc.get_sparse_core_info()
NC = _info.num_cores
NS = _info.num_subcores
L = _info.num_lanes
NW = NC * NS
BPW = B // NW
CH = 128
NCH = BPW // CH
DU = 8


def _make_sc_kernel():
    mesh = plsc.VectorSubcoreMesh(core_axis_name="c", subcore_axis_name="s")

    @functools.partial(
        pl.kernel,
        mesh=mesh,
        compiler_params=pltpu.CompilerParams(needs_layout_passes=False,
                                             use_tc_tiling_on_sc=True),
        out_type=jax.ShapeDtypeStruct((B,), jnp.float32),
        scratch_types=[
            pltpu.VMEM((D, NUM_REL), jnp.float32),
            pltpu.VMEM((BPW,), jnp.int32),
            pltpu.VMEM((D, CH), jnp.float32),
            pltpu.VMEM((D, CH), jnp.float32),
            pltpu.VMEM((D, CH), jnp.float32),
            pltpu.VMEM((D, CH), jnp.float32),
            pltpu.VMEM((BPW,), jnp.float32),
            pltpu.VMEM_SHARED((D, NUM_REL), jnp.float32),
            pltpu.SemaphoreType.DMA,
            pltpu.SemaphoreType.DMA,
        ],
    )
    def sc_kernel(sT_hbm, oT_hbm, rel_hbm, tT_hbm, out_hbm,
                  tT_v, idx_v, s0_v, o0_v, s1_v, o1_v, out_v,
                  shared_v, sem_b0, sem_b1):
        sid = lax.axis_index("s")
        wid = sid * NC + lax.axis_index("c")
        base = wid * BPW
        bufs = ((s0_v, o0_v, sem_b0), (s1_v, o1_v, sem_b1))

        def prefetch(c):
            s_v, o_v, sem = bufs[c % 2]
            off = base + c * CH
            return (pltpu.async_copy(sT_hbm.at[:, pl.ds(off, CH)], s_v, sem),
                    pltpu.async_copy(oT_hbm.at[:, pl.ds(off, CH)], o_v, sem))

        pending = prefetch(0)
        pltpu.sync_copy(rel_hbm.at[pl.ds(base, BPW)], idx_v)

        @pl.when(sid == 0)
        def _():
            pltpu.sync_copy(tT_hbm, shared_v)
        plsc.subcore_barrier()
        pltpu.sync_copy(shared_v, tT_v)

        for c in range(NCH):
            s_v, o_v, _ = bufs[c % 2]
            nxt = prefetch(c + 1) if c + 1 < NCH else ()
            for h in pending:
                h.wait()
            pending = nxt

            def group_body(g, _, c=c, s_v=s_v, o_v=o_v):
                idx16 = idx_v[pl.ds(c * CH + g * L, L)]

                def dq_body(dq, accs, g=g, idx16=idx16, s_v=s_v, o_v=o_v):
                    d0 = dq * DU
                    accs = list(accs)
                    for j in range(DU):
                        sv = s_v[d0 + j, pl.ds(g * L, L)]
                        ov = o_v[d0 + j, pl.ds(g * L, L)]
                        rv = plsc.load_gather(
                            tT_v, [jnp.full((L,), d0 + j, jnp.int32), idx16])
                        accs[j % 4] = accs[j % 4] + sv * rv * ov
                    return tuple(accs)

                z = jnp.zeros((L,), jnp.float32)
                a0, a1, a2, a3 = lax.fori_loop(0, D // DU, dq_body,
                                               (z, z, z, z))
                out_v[pl.ds(c * CH + g * L, L)] = (a0 + a1) + (a2 + a3)
                return 0

            lax.fori_loop(0, CH // L, group_body, 0)

        pltpu.sync_copy(out_v, out_hbm.at[pl.ds(base, BPW)])

    return sc_kernel


_sc_kernel = _make_sc_kernel()


def kernel(subject_embeddings, object_embeddings, relations, relation_weight):
    scores = _sc_kernel(subject_embeddings.T, object_embeddings.T,
                        relations.astype(jnp.int32), relation_weight.T)
    return scores.reshape(B, 1)


# speedup vs baseline: 1.0244x; 1.0244x over previous
import functools

import jax
import jax.numpy as jnp
from jax import lax
from jax.experimental import pallas as pl
from jax.experimental.pallas import tpu as pltpu
from jax.experimental.pallas import tpu_sc as plsc

B = 16384
D = 64
NUM_REL = 1000
TP = 1000

_info = plsc.get_sparse_core_info()
NC = _info.num_cores
NS = _info.num_subcores
L = _info.num_lanes
NW = NC * NS
BPW = B // NW
CH = 128
NCH = BPW // CH
DU = 8


def _make_sc_kernel():
    mesh = plsc.VectorSubcoreMesh(core_axis_name="c", subcore_axis_name="s")

    @functools.partial(
        pl.kernel,
        mesh=mesh,
        compiler_params=pltpu.CompilerParams(needs_layout_passes=False,
                                             use_tc_tiling_on_sc=True),
        out_type=jax.ShapeDtypeStruct((B,), jnp.float32),
        scratch_types=[
            pltpu.VMEM((D * TP,), jnp.float32),
            pltpu.VMEM((BPW,), jnp.int32),
            pltpu.VMEM((D, CH), jnp.float32),
            pltpu.VMEM((D, CH), jnp.float32),
            pltpu.VMEM((D, CH), jnp.float32),
            pltpu.VMEM((D, CH), jnp.float32),
            pltpu.VMEM((BPW,), jnp.float32),
            pltpu.VMEM_SHARED((D * TP,), jnp.float32),
            pltpu.SemaphoreType.DMA,
            pltpu.SemaphoreType.DMA,
        ],
    )
    def sc_kernel(sT_hbm, oT_hbm, rel_hbm, tflat_hbm, out_hbm,
                  tflat_v, idx_v, s0_v, o0_v, s1_v, o1_v, out_v,
                  shared_v, sem_b0, sem_b1):
        sid = lax.axis_index("s")
        wid = sid * NC + lax.axis_index("c")
        base = wid * BPW
        bufs = ((s0_v, o0_v, sem_b0), (s1_v, o1_v, sem_b1))

        def prefetch(c):
            s_v, o_v, sem = bufs[c % 2]
            off = base + c * CH
            return (pltpu.async_copy(sT_hbm.at[:, pl.ds(off, CH)], s_v, sem),
                    pltpu.async_copy(oT_hbm.at[:, pl.ds(off, CH)], o_v, sem))

        pending = prefetch(0)
        pltpu.sync_copy(rel_hbm.at[pl.ds(base, BPW)], idx_v)

        @pl.when(sid == 0)
        def _():
            pltpu.sync_copy(tflat_hbm, shared_v)
        plsc.subcore_barrier()
        pltpu.sync_copy(shared_v, tflat_v)

        for c in range(NCH):
            s_v, o_v, _ = bufs[c % 2]
            nxt = prefetch(c + 1) if c + 1 < NCH else ()
            for h in pending:
                h.wait()
            pending = nxt

            def group_body(g, _, c=c, s_v=s_v, o_v=o_v):
                idx16 = idx_v[pl.ds(c * CH + g * L, L)]

                def dq_body(dq, accs, g=g, idx16=idx16, s_v=s_v, o_v=o_v):
                    d0 = dq * DU
                    accs = list(accs)
                    for j in range(DU):
                        sv = s_v[d0 + j, pl.ds(g * L, L)]
                        ov = o_v[d0 + j, pl.ds(g * L, L)]
                        rv = plsc.load_gather(tflat_v, [idx16 + (d0 + j) * TP])
                        accs[j % 4] = accs[j % 4] + sv * rv * ov
                    return tuple(accs)

                z = jnp.zeros((L,), jnp.float32)
                a0, a1, a2, a3 = lax.fori_loop(0, D // DU, dq_body,
                                               (z, z, z, z))
                out_v[pl.ds(c * CH + g * L, L)] = (a0 + a1) + (a2 + a3)
                return 0

            lax.fori_loop(0, CH // L, group_body, 0)

        pltpu.sync_copy(out_v, out_hbm.at[pl.ds(base, BPW)])

    return sc_kernel


_sc_kernel = _make_sc_kernel()


def kernel(subject_embeddings, object_embeddings, relations, relation_weight):
    tflat = relation_weight.T.reshape(D * TP)
    scores = _sc_kernel(subject_embeddings.T, object_embeddings.T,
                        relations.astype(jnp.int32), tflat)
    return scores.reshape(B, 1)
